# baseline (speedup 1.0000x reference)
import jax
import jax.numpy as jnp
from jax.experimental import pallas as pl

_N = 10000
_K = 32
_CI = 128
_CO = 128
_ND = 3
_NL = 3
_NM = 3
_P = _ND * _NL * _NM
_EPS = 1e-8


def _knn(pos, k=_K, chunk=1000):
    n = pos.shape[0]
    sq = jnp.sum(pos * pos, axis=1)

    def body(pc):
        d2 = jnp.sum(pc * pc, axis=1)[:, None] - 2.0 * pc @ pos.T + sq[None, :]
        return jax.lax.top_k(-d2, k)[1]

    chunks = pos.reshape(n // chunk, chunk, 3)
    return jax.lax.map(body, chunks).reshape(n, k)


def _mix_kernel(a_ref, w_ref, o_ref):
    o_ref[...] = jnp.dot(a_ref[...], w_ref[...],
                         preferred_element_type=jnp.float32)


def kernel(position_matrix, channel_matrix, W):
    nbr = _knn(jax.lax.stop_gradient(position_matrix), _K)

    nbr_pos = position_matrix[nbr]
    centers = jnp.mean(nbr_pos, axis=1)
    local = nbr_pos - centers[:, None, :]
    cov = jnp.einsum('nki,nkj->nij', local, local) / _K
    _, eigvecs = jnp.linalg.eigh(cov)
    eigvecs = eigvecs[:, :, ::-1]
    lp = jnp.einsum('nki,nij->nkj', local, eigvecs)
    x, y, z = lp[..., 0], lp[..., 1], lp[..., 2]
    r = jnp.sqrt(x * x + y * y + z * z + _EPS)
    ct = jnp.clip(z / r, -1.0 + 1e-6, 1.0 - 1e-6)
    theta = jnp.arccos(ct)
    phi = jnp.arctan2(y, x + _EPS)

    rn = r / (jnp.max(r, axis=1, keepdims=True) + _EPS)
    Br = rn[..., None] ** jnp.arange(_ND, dtype=jnp.float32)
    Bt = jnp.cos(theta[..., None] * jnp.arange(_NL, dtype=jnp.float32))
    Bp = jnp.cos(phi[..., None] * jnp.arange(_NM, dtype=jnp.float32))
    B = jnp.einsum('nka,nkb,nkc->nkabc', Br, Bt, Bp).reshape(_N, _K, _P)

    feat = channel_matrix[nbr]
    Mnp = jnp.einsum('nkp,nki->npi', B, feat)

    A = Mnp.reshape(_N, _P * _CI)
    Wf = W.reshape(_P * _CI, _CO)
    RB = 400
    out = pl.pallas_call(
        _mix_kernel,
        grid=(_N // RB,),
        in_specs=[
            pl.BlockSpec((RB, _P * _CI), lambda i: (i, 0)),
            pl.BlockSpec((_P * _CI, _CO), lambda i: (0, 0)),
        ],
        out_specs=pl.BlockSpec((RB, _CO), lambda i: (i, 0)),
        out_shape=jax.ShapeDtypeStruct((_N, _CO), jnp.float32),
    )(A, Wf)
    return out


# speedup vs baseline: 9.0952x; 9.0952x over previous
import jax
import jax.numpy as jnp
from jax.experimental import pallas as pl

_N = 10000
_K = 32
_CI = 128
_CO = 128
_P = 27
_EPS = 1e-8
_NPAD = 79 * 128
_QB = 200
_GB = 200
_BIG = 3e38
_IBIG = 2 ** 30
_SWEEPS = 6


def _topk_body(d_ref, o_ref):
    rows = _NPAD // 128
    d3 = d_ref[...].reshape(_QB, rows, 128)

    row_iota = jax.lax.broadcasted_iota(jnp.int32, (_QB, rows, 128), 1)
    cand_v = []
    cand_r = []
    for _ in range(8):
        m = jnp.min(d3, axis=1)
        eq = d3 == m[:, None, :]
        ridx = jnp.min(jnp.where(eq, row_iota, _IBIG), axis=1)
        cand_v.append(m)
        cand_r.append(ridx)
        d3 = jnp.where(row_iota == ridx[:, None, :], _BIG, d3)

    lane_iota = jax.lax.broadcasted_iota(jnp.int32, (_QB, 8, 128), 2)
    cv = jnp.stack(cand_v, axis=1).reshape(_QB, 8 * 128)
    cg = (jnp.stack(cand_r, axis=1) * 128 + lane_iota).reshape(_QB, 8 * 128)

    col_iota = jax.lax.broadcasted_iota(jnp.int32, (_QB, 8 * 128), 1)
    for it in range(_K):
        m = jnp.min(cv, axis=1, keepdims=True)
        eq = cv == m
        g = jnp.min(jnp.where(eq, cg, _IBIG), axis=1)
        o_ref[:, it] = g
        chose = eq & (cg == g[:, None])
        pos = jnp.min(jnp.where(chose, col_iota, _IBIG), axis=1, keepdims=True)
        cv = jnp.where(col_iota == pos, _BIG, cv)


def _topk_pallas(d2p):
    return pl.pallas_call(
        _topk_body,
        grid=(_N // _QB,),
        in_specs=[pl.BlockSpec((_QB, _NPAD), lambda i: (i, 0))],
        out_specs=pl.BlockSpec((_QB, _K), lambda i: (i, 0)),
        out_shape=jax.ShapeDtypeStruct((_N, _K), jnp.int32),
    )(d2p)


def _jacobi_rotation(a, p, q):
    app = a[(p, p)]
    aqq = a[(q, q)]
    apq = a[(p, q)]
    tau = (aqq - app) / (2.0 * apq)
    sq = jnp.sqrt(1.0 + tau * tau)
    t = jnp.where(tau >= 0, 1.0 / (tau + sq), -1.0 / (-tau + sq))
    t = jnp.where(jnp.abs(apq) < 1e-37, 0.0, t)
    c = 1.0 / jnp.sqrt(1.0 + t * t)
    s = t * c
    return c, s


def _jacobi_apply(a, v, p, q, c, s):
    r = 3 - p - q
    app = a[(p, p)]; aqq = a[(q, q)]; apq = a[(p, q)]
    apr = a[(min(p, r), max(p, r))]
    aqr = a[(min(q, r), max(q, r))]
    a[(p, p)] = c * c * app - 2.0 * c * s * apq + s * s * aqq
    a[(q, q)] = s * s * app + 2.0 * c * s * apq + c * c * aqq
    a[(p, q)] = (c * c - s * s) * apq + c * s * (app - aqq)
    a[(min(p, r), max(p, r))] = c * apr - s * aqr
    a[(min(q, r), max(q, r))] = s * apr + c * aqr
    for i in range(3):
        vip = v[(i, p)]
        viq = v[(i, q)]
        v[(i, p)] = c * vip - s * viq
        v[(i, q)] = s * vip + c * viq


def _sel3(j, x0, x1, x2):
    return jnp.where(j == 0, x0, jnp.where(j == 1, x1, x2))


def _bf(x):
    return x.astype(jnp.bfloat16).astype(jnp.float32)


def _geom_body(px_ref, py_ref, pz_ref, cov_ref, feat_ref, w_ref, o_ref):
    px = px_ref[...]; py = py_ref[...]; pz = pz_ref[...]
    lx = px - jnp.mean(px, axis=1, keepdims=True)
    ly = py - jnp.mean(py, axis=1, keepdims=True)
    lz = pz - jnp.mean(pz, axis=1, keepdims=True)

    covm = cov_ref[...]
    a = {
        (0, 0): covm[:, 0],
        (0, 1): covm[:, 1],
        (0, 2): covm[:, 2],
        (1, 1): covm[:, 3],
        (1, 2): covm[:, 4],
        (2, 2): covm[:, 5],
    }
    one = jnp.ones_like(a[(0, 0)])
    zero = jnp.zeros_like(one)
    v = {(i, j): (one if i == j else zero) for i in range(3) for j in range(3)}

    for _ in range(_SWEEPS):
        for (p, q) in ((0, 2), (1, 2), (0, 1)):
            c, s = _jacobi_rotation(a, p, q)
            _jacobi_apply(a, v, p, q, c, s)

    w0 = a[(0, 0)]; w1 = a[(1, 1)]; w2 = a[(2, 2)]
    mx = jnp.maximum(jnp.maximum(w0, w1), w2)
    j0 = jnp.where(w2 == mx, 2, jnp.where(w1 == mx, 1, 0))
    mn = jnp.minimum(jnp.minimum(w0, w1), w2)
    j2 = jnp.where(w0 == mn, 0, jnp.where(w1 == mn, 1, 2))
    j1 = 3 - j0 - j2

    def col(j, i):
        return _bf(_sel3(j, v[(i, 0)], v[(i, 1)], v[(i, 2)]))[:, None]

    bx, by, bz = _bf(lx), _bf(ly), _bf(lz)
    x = bx * col(j0, 0) + by * col(j0, 1) + bz * col(j0, 2)
    y = bx * col(j1, 0) + by * col(j1, 1) + bz * col(j1, 2)
    z = bx * col(j2, 0) + by * col(j2, 1) + bz * col(j2, 2)

    r = jnp.sqrt(x * x + y * y + z * z + _EPS)
    ct = jnp.clip(z / r, -1.0 + 1e-6, 1.0 - 1e-6)
    xe = x + _EPS
    h = jnp.sqrt(xe * xe + y * y)
    cp = jnp.where(h > 0.0, xe / jnp.where(h > 0.0, h, 1.0), 1.0)

    rn = r / (jnp.max(r, axis=1, keepdims=True) + _EPS)
    br = (jnp.ones_like(rn), rn, rn * rn)
    bt = (jnp.ones_like(ct), ct, 2.0 * ct * ct - 1.0)
    bp = (jnp.ones_like(cp), cp, 2.0 * cp * cp - 1.0)

    feat = feat_ref[...].reshape(_GB, _K, _CI)
    pieces = []
    for ia in range(3):
        for ib in range(3):
            for ic in range(3):
                wgt = br[ia] * bt[ib] * bp[ic]
                pieces.append(jnp.sum(wgt[:, :, None] * feat, axis=1))
    A = jnp.concatenate(pieces, axis=1)
    o_ref[...] = jnp.dot(A, w_ref[...], preferred_element_type=jnp.float32,
                         precision=jax.lax.Precision.HIGHEST)


def _geom_pallas(px, py, pz, covm, feat, Wf):
    return pl.pallas_call(
        _geom_body,
        grid=(_N // _GB,),
        in_specs=[
            pl.BlockSpec((_GB, _K), lambda i: (i, 0)),
            pl.BlockSpec((_GB, _K), lambda i: (i, 0)),
            pl.BlockSpec((_GB, _K), lambda i: (i, 0)),
            pl.BlockSpec((_GB, 8), lambda i: (i, 0)),
            pl.BlockSpec((_GB * _K, _CI), lambda i: (i, 0)),
            pl.BlockSpec((_P * _CI, _CO), lambda i: (0, 0)),
        ],
        out_specs=pl.BlockSpec((_GB, _CO), lambda i: (i, 0)),
        out_shape=jax.ShapeDtypeStruct((_N, _CO), jnp.float32),
    )(px, py, pz, covm, feat, Wf)


def kernel(position_matrix, channel_matrix, W):
    pos = position_matrix
    sq = jnp.sum(pos * pos, axis=1)
    d2 = sq[:, None] - 2.0 * pos @ pos.T + sq[None, :]
    d2p = jnp.pad(d2, ((0, 0), (0, _NPAD - _N)), constant_values=_BIG)
    nbr = _topk_pallas(d2p)

    nbr_pos = pos[nbr]
    centers = jnp.mean(nbr_pos, axis=1)
    local = nbr_pos - centers[:, None, :]
    cov = jnp.einsum('nki,nkj->nij', local, local) / _K
    covm = jnp.stack([cov[:, 0, 0], cov[:, 0, 1], cov[:, 0, 2],
                      cov[:, 1, 1], cov[:, 1, 2], cov[:, 2, 2],
                      cov[:, 0, 0], cov[:, 0, 0]], axis=1)

    px = nbr_pos[..., 0]
    py = nbr_pos[..., 1]
    pz = nbr_pos[..., 2]
    feat = channel_matrix[nbr].reshape(_N * _K, _CI)
    Wf = W.reshape(_P * _CI, _CO)
    return _geom_pallas(px, py, pz, covm, feat, Wf)
